# baseline (speedup 1.0000x reference)
import functools

import jax
import jax.numpy as jnp
from jax import lax
from jax.experimental import pallas as pl
from jax.experimental.pallas import tpu as pltpu
from jax.experimental.pallas import tpu_sc as plsc

_NUM_EXPERTS = 16
_BLOCK_T = 2048
_H_SPLIT = 4


def _probs_kernel(*refs):
    x_refs = refs[:_H_SPLIT]
    w_ref, b_ref = refs[_H_SPLIT], refs[_H_SPLIT + 1]
    probs_ref = refs[_H_SPLIT + 2]

    w = w_ref[...]
    hc = w.shape[1] // _H_SPLIT
    logits = b_ref[...]
    for j in range(_H_SPLIT):
        logits = logits + jax.lax.dot_general(
            x_refs[j][...], w[:, j * hc:(j + 1) * hc],
            (((1,), (1,)), ((), ())), preferred_element_type=jnp.float32,
        )
    m = jnp.max(logits, axis=-1, keepdims=True)
    e = jnp.exp(logits - m)
    probs_ref[...] = e / jnp.sum(e, axis=-1, keepdims=True)


def _tc_probs(x, W, b2):
    n_tokens, hidden_dim = x.shape
    hc = hidden_dim // _H_SPLIT
    grid = (n_tokens // _BLOCK_T,)
    x_specs = [
        pl.BlockSpec((_BLOCK_T, hc), lambda i, j=j: (i, j)) for j in range(_H_SPLIT)
    ]
    return pl.pallas_call(
        _probs_kernel,
        grid=grid,
        in_specs=x_specs + [
            pl.BlockSpec((_NUM_EXPERTS, hidden_dim), lambda i: (0, 0)),
            pl.BlockSpec((1, _NUM_EXPERTS), lambda i: (0, 0)),
        ],
        out_specs=pl.BlockSpec((_BLOCK_T, _NUM_EXPERTS), lambda i: (i, 0)),
        out_shape=jax.ShapeDtypeStruct((n_tokens, _NUM_EXPERTS), jnp.float32),
    )(*([x] * _H_SPLIT), W, b2)


def _make_sc_routing(n_tokens):
    info = plsc.get_sparse_core_info()
    nc, ns = info.num_cores, info.num_subcores
    nw = nc * ns
    t_per_w = n_tokens // nw
    mesh = plsc.VectorSubcoreMesh(core_axis_name="c", subcore_axis_name="s")

    e_per_w = t_per_w * _NUM_EXPERTS

    @functools.partial(
        pl.kernel, mesh=mesh,
        out_type=[
            jax.ShapeDtypeStruct((n_tokens * _NUM_EXPERTS,), jnp.float32),
            jax.ShapeDtypeStruct((n_tokens * _NUM_EXPERTS,), jnp.int32),
            jax.ShapeDtypeStruct((n_tokens * _NUM_EXPERTS,), jnp.float32),
        ],
        scratch_types=[
            pltpu.VMEM((e_per_w,), jnp.float32),
            pltpu.VMEM((e_per_w,), jnp.float32),
            pltpu.VMEM((e_per_w,), jnp.int32),
            pltpu.VMEM((e_per_w,), jnp.float32),
        ],
    )
    def sc_routing(probs_hbm, rw_hbm, idx_hbm, t2p_hbm, probs_v, rw_v, idx_v, t2p_v):
        wid = lax.axis_index("s") * nc + lax.axis_index("c")
        base = wid * e_per_w
        pltpu.sync_copy(probs_hbm.at[pl.ds(base, e_per_w)], probs_v)

        iota = lax.broadcasted_iota(jnp.int32, (16,), 0)
        perms = [iota ^ sh for sh in (1, 2, 4, 8)]

        gdn = lax.GatherDimensionNumbers(
            offset_dims=(), collapsed_slice_dims=(0,), start_index_map=(0,))

        def _perm(u, p):
            return lax.gather(u, p[:, None], gdn, slice_sizes=(1,),
                              mode=lax.GatherScatterMode.PROMISE_IN_BOUNDS)

        def _allmax(u):
            for p in perms:
                u = jnp.maximum(u, _perm(u, p))
            return u

        def _allmin(u):
            for p in perms:
                u = jnp.minimum(u, _perm(u, p))
            return u

        def body(t, carry):
            v = probs_v[pl.ds(t * _NUM_EXPERTS, _NUM_EXPERTS)]
            p1 = _allmax(v)
            i1 = _allmin(jnp.where(v == p1, iota, _NUM_EXPERTS))
            masked = jnp.where(iota == i1, -jnp.inf, v)
            p2 = _allmax(masked)
            i2 = _allmin(jnp.where(masked == p2, iota, _NUM_EXPERTS))
            s = p1 + p2
            p1n = p1 / s
            p2n = p2 / s
            sl = pl.ds(t * _NUM_EXPERTS, _NUM_EXPERTS)
            rw_v[sl] = jnp.where(iota == i1, p1n,
                                 jnp.where(iota == i2, p2n, 0.0))
            idx_v[sl] = jnp.where(iota == 0, i1, jnp.where(iota == 1, i2, 0))
            t2p_v[sl] = jnp.where(iota == 0, p1n, jnp.where(iota == 1, p2n, 0.0))
            return carry

        lax.fori_loop(0, t_per_w, body, 0)
        pltpu.sync_copy(rw_v, rw_hbm.at[pl.ds(base, e_per_w)])
        pltpu.sync_copy(idx_v, idx_hbm.at[pl.ds(base, e_per_w)])
        pltpu.sync_copy(t2p_v, t2p_hbm.at[pl.ds(base, e_per_w)])

    return sc_routing


@jax.jit
def kernel(hidden_states, W, b):
    batch_size, seq_len, hidden_dim = hidden_states.shape
    n_tokens = batch_size * seq_len
    x = hidden_states.reshape(n_tokens, hidden_dim)
    b2 = b.reshape(1, _NUM_EXPERTS)

    router_probs = _tc_probs(x, W, b2)
    rw, idx16, t2p16 = _make_sc_routing(n_tokens)(
        router_probs.reshape(n_tokens * _NUM_EXPERTS))
    rw = rw.reshape(n_tokens, _NUM_EXPERTS)
    idx16 = idx16.reshape(n_tokens, _NUM_EXPERTS)
    t2p16 = t2p16.reshape(n_tokens, _NUM_EXPERTS)
    return (rw, idx16[:, :2], router_probs, t2p16[:, :2])


# speedup vs baseline: 1.4652x; 1.4652x over previous
import jax
import jax.numpy as jnp
from jax.experimental import pallas as pl

_NUM_EXPERTS = 16
_BLOCK_T = 2048


def _routing_kernel(x_ref, w_ref, b_ref, rw_ref, idx_ref, probs_ref, top2p_ref):
    x = x_ref[...]
    w = w_ref[...]
    logits = jax.lax.dot_general(
        x, w, (((1,), (1,)), ((), ())), preferred_element_type=jnp.float32
    ) + b_ref[...]

    m = jnp.max(logits, axis=-1, keepdims=True)
    e = jnp.exp(logits - m)
    denom = jnp.sum(e, axis=-1, keepdims=True)
    probs = e / denom
    p1 = 1.0 / denom

    lane = jax.lax.broadcasted_iota(jnp.int32, probs.shape, 1)
    i1 = jnp.min(jnp.where(probs == p1, lane, _NUM_EXPERTS), axis=-1, keepdims=True)
    masked = jnp.where(lane == i1, -jnp.inf, probs)
    p2 = jnp.max(masked, axis=-1, keepdims=True)
    i2 = jnp.min(jnp.where(masked == p2, lane, _NUM_EXPERTS), axis=-1, keepdims=True)

    s = p1 + p2
    p1n = p1 / s
    p2n = p2 / s

    rw_ref[...] = jnp.where(lane == i1, p1n, jnp.where(lane == i2, p2n, 0.0))
    probs_ref[...] = probs
    idx_ref[...] = jnp.concatenate([i1, i2], axis=-1)
    top2p_ref[...] = jnp.concatenate([p1n, p2n], axis=-1)


@jax.jit
def kernel(hidden_states, W, b):
    batch_size, seq_len, hidden_dim = hidden_states.shape
    n_tokens = batch_size * seq_len
    x = hidden_states.reshape(n_tokens, hidden_dim)
    b2 = b.reshape(1, _NUM_EXPERTS)

    grid = (n_tokens // _BLOCK_T,)
    out = pl.pallas_call(
        _routing_kernel,
        grid=grid,
        in_specs=[
            pl.BlockSpec((_BLOCK_T, hidden_dim), lambda i: (i, 0)),
            pl.BlockSpec((_NUM_EXPERTS, hidden_dim), lambda i: (0, 0)),
            pl.BlockSpec((1, _NUM_EXPERTS), lambda i: (0, 0)),
        ],
        out_specs=[
            pl.BlockSpec((_BLOCK_T, _NUM_EXPERTS), lambda i: (i, 0)),
            pl.BlockSpec((_BLOCK_T, 2), lambda i: (i, 0)),
            pl.BlockSpec((_BLOCK_T, _NUM_EXPERTS), lambda i: (i, 0)),
            pl.BlockSpec((_BLOCK_T, 2), lambda i: (i, 0)),
        ],
        out_shape=[
            jax.ShapeDtypeStruct((n_tokens, _NUM_EXPERTS), jnp.float32),
            jax.ShapeDtypeStruct((n_tokens, 2), jnp.int32),
            jax.ShapeDtypeStruct((n_tokens, _NUM_EXPERTS), jnp.float32),
            jax.ShapeDtypeStruct((n_tokens, 2), jnp.float32),
        ],
    )(x, W, b2)
    routing_weights, top2_indices, router_probs, top2_probs = out
    return (routing_weights, top2_indices, router_probs, top2_probs)
